# Initial kernel scaffold; baseline (speedup 1.0000x reference)
import functools

import jax
import jax.numpy as jnp
from jax import lax
from jax.experimental import pallas as pl
from jax.experimental.pallas import tpu as pltpu
from jax.experimental.pallas import tpu_sc as plsc

D = 128
NC = 2
NS = 16
NW = NC * NS
CHUNK = 128


def _make_gather(batch_total: int):
    b_per_w = batch_total // NW
    nchunk = b_per_w // CHUNK
    mesh = plsc.VectorSubcoreMesh(core_axis_name="c", subcore_axis_name="s")

    @functools.partial(
        pl.kernel,
        out_type=jax.ShapeDtypeStruct((batch_total, D), jnp.float32),
        mesh=mesh,
        scratch_types=[
            pltpu.VMEM((nchunk, CHUNK), jnp.int32),
            pltpu.VMEM((CHUNK, D), jnp.float32),
            pltpu.SemaphoreType.DMA,
        ],
    )
    def gather_kernel(idx_hbm, table_hbm, out_hbm, idx_v, buf, gsem):
        wid = lax.axis_index("s") * NC + lax.axis_index("c")
        base = wid * b_per_w
        pltpu.sync_copy(idx_hbm.at[pl.ds(wid * nchunk, nchunk)], idx_v)

        def chunk_body(j, carry):
            pltpu.async_copy(table_hbm.at[idx_v.at[j]], buf, gsem).wait()
            pltpu.sync_copy(buf, out_hbm.at[pl.ds(base + j * CHUNK, CHUNK)])
            return carry

        lax.fori_loop(0, nchunk, chunk_body, 0, unroll=False)

    return gather_kernel


def kernel(indices, W):
    batch, seq = indices.shape
    total = batch * seq
    idx2d = indices.reshape(total // CHUNK, CHUNK)
    out = _make_gather(total)(idx2d, W)
    return out.reshape(batch, seq, D)


# speedup vs baseline: 2.9729x; 2.9729x over previous
import functools

import jax
import jax.numpy as jnp
from jax import lax
from jax.experimental import pallas as pl
from jax.experimental.pallas import tpu as pltpu
from jax.experimental.pallas import tpu_sc as plsc

D = 128
NC = 2
NS = 16
NW = NC * NS
CHUNK = 128


def _make_gather(batch_total: int):
    b_per_w = batch_total // NW
    nchunk = b_per_w // CHUNK
    mesh = plsc.VectorSubcoreMesh(core_axis_name="c", subcore_axis_name="s")

    @functools.partial(
        pl.kernel,
        out_type=jax.ShapeDtypeStruct((batch_total, D), jnp.float32),
        mesh=mesh,
        scratch_types=[
            pltpu.VMEM((nchunk, CHUNK), jnp.int32),
            pltpu.VMEM((CHUNK, D), jnp.float32),
            pltpu.SemaphoreType.DMA,
        ],
    )
    def gather_kernel(idx_hbm, table_hbm, out_hbm, idx_v, buf, gsem):
        wid = lax.axis_index("s") * NC + lax.axis_index("c")
        base = wid * b_per_w
        pltpu.sync_copy(idx_hbm.at[wid], idx_v)

        def chunk_body(j, carry):
            pltpu.async_copy(table_hbm.at[idx_v.at[j]], buf, gsem).wait()
            start = pl.multiple_of(base + j * CHUNK, CHUNK)
            pltpu.sync_copy(buf, out_hbm.at[pl.ds(start, CHUNK)])
            return carry

        lax.fori_loop(0, nchunk, chunk_body, 0, unroll=False)

    return gather_kernel


def kernel(indices, W):
    batch, seq = indices.shape
    total = batch * seq
    idx2d = indices.reshape(NW, total // (NW * CHUNK), CHUNK)
    out = _make_gather(total)(idx2d, W)
    return out.reshape(batch, seq, D)


# speedup vs baseline: 3.3179x; 1.1160x over previous
import functools

import jax
import jax.numpy as jnp
from jax import lax
from jax.experimental import pallas as pl
from jax.experimental.pallas import tpu as pltpu
from jax.experimental.pallas import tpu_sc as plsc

D = 128
NC = 2
NS = 16
NW = NC * NS
CHUNK = 128
NBUF = 5


def _make_gather(batch_total: int):
    b_per_w = batch_total // NW
    nchunk = b_per_w // CHUNK
    ngroups = nchunk // NBUF
    assert nchunk % NBUF == 0
    mesh = plsc.VectorSubcoreMesh(core_axis_name="c", subcore_axis_name="s")

    @functools.partial(
        pl.kernel,
        out_type=jax.ShapeDtypeStruct((batch_total, D), jnp.float32),
        mesh=mesh,
        scratch_types=[
            pltpu.VMEM((nchunk, CHUNK), jnp.int32),
            pltpu.VMEM((NBUF, CHUNK, D), jnp.float32),
            [pltpu.SemaphoreType.DMA] * NBUF,
            [pltpu.SemaphoreType.DMA] * NBUF,
        ],
    )
    def gather_kernel(idx_hbm, table_hbm, out_hbm, idx_v, bufs, gsems, ssems):
        wid = lax.axis_index("s") * NC + lax.axis_index("c")
        base = wid * b_per_w
        pltpu.sync_copy(idx_hbm.at[wid], idx_v)

        def start_gather(j, b):
            pltpu.async_copy(table_hbm.at[idx_v.at[j]], bufs.at[b], gsems[b])

        def wait_gather(b):
            pltpu.make_async_copy(
                table_hbm.at[idx_v.at[0]], bufs.at[b], gsems[b]
            ).wait()

        def out_slice(j):
            start = pl.multiple_of(base + j * CHUNK, CHUNK)
            return out_hbm.at[pl.ds(start, CHUNK)]

        def start_store(j, b):
            pltpu.async_copy(bufs.at[b], out_slice(j), ssems[b])

        def wait_store(b):
            pltpu.make_async_copy(bufs.at[b], out_slice(0), ssems[b]).wait()

        for b in range(NBUF):
            start_gather(b, b)

        def group_body(g, carry):
            j0 = g * NBUF
            for b in range(NBUF):
                wait_gather(b)
                start_store(j0 + b, b)
            for b in range(NBUF):
                wait_store(b)
                start_gather(j0 + NBUF + b, b)
            return carry

        lax.fori_loop(0, ngroups - 1, group_body, 0, unroll=False)

        j0 = (ngroups - 1) * NBUF
        for b in range(NBUF):
            wait_gather(b)
            start_store(j0 + b, b)
        for b in range(NBUF):
            wait_store(b)

    return gather_kernel


def kernel(indices, W):
    batch, seq = indices.shape
    total = batch * seq
    idx2d = indices.reshape(NW, total // (NW * CHUNK), CHUNK)
    out = _make_gather(total)(idx2d, W)
    return out.reshape(batch, seq, D)


# speedup vs baseline: 5.9474x; 1.7925x over previous
import functools

import jax
import jax.numpy as jnp
from jax import lax
from jax.experimental import pallas as pl
from jax.experimental.pallas import tpu as pltpu
from jax.experimental.pallas import tpu_sc as plsc

D = 128
NC = 2
NS = 16
NW = NC * NS
NBUF = 8


def _make_gather(batch: int, seq: int):
    b_per_w = batch // NW
    ngroups = b_per_w // NBUF
    assert b_per_w % NBUF == 0
    mesh = plsc.VectorSubcoreMesh(core_axis_name="c", subcore_axis_name="s")

    @functools.partial(
        pl.kernel,
        out_type=jax.ShapeDtypeStruct((batch, seq, D), jnp.float32),
        mesh=mesh,
        scratch_types=[
            pltpu.VMEM((b_per_w, seq), jnp.int32),
            pltpu.VMEM((NBUF, seq, D), jnp.float32),
            [pltpu.SemaphoreType.DMA] * NBUF,
            [pltpu.SemaphoreType.DMA] * NBUF,
        ],
    )
    def gather_kernel(idx_hbm, table_hbm, out_hbm, idx_v, bufs, gsems, ssems):
        wid = lax.axis_index("s") * NC + lax.axis_index("c")
        base = wid * b_per_w
        pltpu.sync_copy(idx_hbm.at[pl.ds(base, b_per_w)], idx_v)

        def start_gather(r, b):
            pltpu.async_copy(table_hbm.at[idx_v.at[r]], bufs.at[b], gsems[b])

        def wait_gather(b):
            pltpu.make_async_copy(
                table_hbm.at[idx_v.at[0]], bufs.at[b], gsems[b]
            ).wait()

        def start_store(r, b):
            pltpu.async_copy(bufs.at[b], out_hbm.at[base + r], ssems[b])

        def wait_store(b):
            pltpu.make_async_copy(bufs.at[b], out_hbm.at[0], ssems[b]).wait()

        for b in range(NBUF):
            start_gather(b, b)

        def group_body(g, carry):
            r0 = g * NBUF
            for b in range(NBUF):
                wait_gather(b)
                start_store(r0 + b, b)
            for b in range(NBUF):
                wait_store(b)
                start_gather(r0 + NBUF + b, b)
            return carry

        lax.fori_loop(0, ngroups - 1, group_body, 0, unroll=False)

        r0 = (ngroups - 1) * NBUF
        for b in range(NBUF):
            wait_gather(b)
            start_store(r0 + b, b)
        for b in range(NBUF):
            wait_store(b)

    return gather_kernel


def kernel(indices, W):
    batch, seq = indices.shape
    return _make_gather(batch, seq)(indices, W)
